# Initial kernel scaffold; baseline (speedup 1.0000x reference)
import functools
import math

import numpy as np
import jax
import jax.numpy as jnp
from jax import lax
from jax.experimental import pallas as pl
from jax.experimental.pallas import tpu as pltpu
from jax.experimental.pallas import tpu_sc as plsc

D_MODEL = 64
_SCALE = 8.0
_NLANES = 16


def _pe_np(seq: int) -> np.ndarray:
    pe = np.zeros((seq, D_MODEL), dtype=np.float32)
    pos = np.arange(seq, dtype=np.float32)[:, None]
    div_term = np.exp(
        np.arange(0, D_MODEL, 2, dtype=np.float32) * (-math.log(10000.0) / D_MODEL)
    )
    pe[:, 0::2] = np.sin(pos * div_term)
    pe[:, 1::2] = np.cos(pos * div_term)
    return pe


@jax.jit
def _run(x_r, table, pe_rep):
    NW, n_chunks, K, IDXW = x_r.shape
    chunk = K * IDXW
    V, D = table.shape
    per_w = n_chunks * chunk
    total = NW * per_w

    mesh = plsc.VectorSubcoreMesh(core_axis_name="c", subcore_axis_name="s")

    @functools.partial(
        pl.kernel,
        out_type=jax.ShapeDtypeStruct((total, D), jnp.float32),
        mesh=mesh,
        scratch_types=[
            pltpu.VMEM((K, IDXW), jnp.int32),
            pltpu.VMEM((chunk, D), jnp.float32),
            pltpu.VMEM((chunk, D), jnp.float32),
            pltpu.SemaphoreType.DMA,
        ],
    )
    def sc_kernel(x_hbm, table_hbm, pe_hbm, out_hbm, idx_v, buf, pe_v, sem):
        wid = lax.axis_index("s") * 2 + lax.axis_index("c")
        pltpu.sync_copy(pe_hbm, pe_v)

        def chunk_body(c, carry):
            pltpu.sync_copy(x_hbm.at[wid, c], idx_v)
            copies = [
                pltpu.async_copy(
                    table_hbm.at[idx_v.at[j]],
                    buf.at[pl.ds(j * IDXW, IDXW)],
                    sem,
                )
                for j in range(K)
            ]
            for cp in copies:
                cp.wait()

            def row_body(r, rc):
                for j in range(D // _NLANES):
                    s = pl.ds(j * _NLANES, _NLANES)
                    buf[r, s] = buf[r, s] * _SCALE + pe_v[r, s]
                return rc

            lax.fori_loop(0, chunk, row_body, 0, unroll=4)

            r0 = wid * per_w + c * chunk
            pltpu.sync_copy(buf, out_hbm.at[pl.ds(r0, chunk)])
            return carry

        lax.fori_loop(0, n_chunks, chunk_body, 0)

    return sc_kernel(x_r, table, pe_rep)


def kernel(x, table):
    B, S = x.shape
    V, D = table.shape
    NW = 32
    CB = 2
    chunk = CB * S
    IDXW = 100
    K = chunk // IDXW
    per_w = (B * S) // NW
    n_chunks = per_w // chunk

    pe_rep = jnp.asarray(np.tile(_pe_np(S), (CB, 1)))
    x_r = x.reshape(NW, n_chunks, K, IDXW)
    out = _run(x_r, table, pe_rep)
    return out.reshape(B, S, D)


# speedup vs baseline: 2.4123x; 2.4123x over previous
import functools
import math

import numpy as np
import jax
import jax.numpy as jnp
from jax import lax
from jax.experimental import pallas as pl
from jax.experimental.pallas import tpu as pltpu
from jax.experimental.pallas import tpu_sc as plsc

D_MODEL = 64
_SCALE = 8.0
_NLANES = 16


def _pe_np(seq: int) -> np.ndarray:
    pe = np.zeros((seq, D_MODEL), dtype=np.float32)
    pos = np.arange(seq, dtype=np.float32)[:, None]
    div_term = np.exp(
        np.arange(0, D_MODEL, 2, dtype=np.float32) * (-math.log(10000.0) / D_MODEL)
    )
    pe[:, 0::2] = np.sin(pos * div_term)
    pe[:, 1::2] = np.cos(pos * div_term)
    return pe


@jax.jit
def _run(x_r, table, pe_rep):
    NW, n_chunks, K, IDXW = x_r.shape
    chunk = K * IDXW
    V, D = table.shape
    per_w = n_chunks * chunk
    total = NW * per_w

    mesh = plsc.VectorSubcoreMesh(core_axis_name="c", subcore_axis_name="s")

    @functools.partial(
        pl.kernel,
        out_type=jax.ShapeDtypeStruct((total, D), jnp.float32),
        mesh=mesh,
        scratch_types=[
            pltpu.VMEM((K, IDXW), jnp.int32),
            pltpu.VMEM((chunk, D), jnp.float32),
            pltpu.VMEM((chunk, D), jnp.float32),
            pltpu.SemaphoreType.DMA,
        ],
        compiler_params=pltpu.CompilerParams(use_tc_tiling_on_sc=False),
    )
    def sc_kernel(x_hbm, table_hbm, pe_hbm, out_hbm, idx_v, buf, pe_v, sem):
        wid = lax.axis_index("s") * 2 + lax.axis_index("c")
        pltpu.sync_copy(pe_hbm, pe_v)

        def chunk_body(c, carry):
            pltpu.sync_copy(x_hbm.at[wid, c], idx_v)
            copies = [
                pltpu.async_copy(
                    table_hbm.at[idx_v.at[j]],
                    buf.at[pl.ds(j * IDXW, IDXW)],
                    sem,
                )
                for j in range(K)
            ]
            for cp in copies:
                cp.wait()

            def row_body(r, rc):
                for j in range(D // _NLANES):
                    s = pl.ds(j * _NLANES, _NLANES)
                    buf[r, s] = buf[r, s] * _SCALE + pe_v[r, s]
                return rc

            lax.fori_loop(0, chunk, row_body, 0, unroll=4)

            r0 = wid * per_w + c * chunk
            pltpu.sync_copy(buf, out_hbm.at[pl.ds(r0, chunk)])
            return carry

        lax.fori_loop(0, n_chunks, chunk_body, 0)

    return sc_kernel(x_r, table, pe_rep)


def kernel(x, table):
    B, S = x.shape
    V, D = table.shape
    NW = 32
    CB = 2
    chunk = CB * S
    IDXW = 100
    K = chunk // IDXW
    per_w = (B * S) // NW
    n_chunks = per_w // chunk

    pe_rep = jnp.asarray(np.tile(_pe_np(S), (CB, 1)))
    x_r = x.reshape(NW, n_chunks, K, IDXW)
    out = _run(x_r, table, pe_rep)
    return out.reshape(B, S, D)


# speedup vs baseline: 3.3811x; 1.4016x over previous
import functools
import math

import numpy as np
import jax
import jax.numpy as jnp
from jax import lax
from jax.experimental import pallas as pl
from jax.experimental.pallas import tpu as pltpu
from jax.experimental.pallas import tpu_sc as plsc

D_MODEL = 64
_SCALE = 8.0
_NLANES = 16
_NBUF = 4


@jax.jit
def _run(x_r, table, pe):
    NW, n_chunks, K, IDXW = x_r.shape
    chunk = K * IDXW
    V, D = table.shape
    per_w = n_chunks * chunk
    total = NW * per_w
    n_groups = n_chunks // _NBUF

    mesh = plsc.VectorSubcoreMesh(core_axis_name="c", subcore_axis_name="s")

    @functools.partial(
        pl.kernel,
        out_type=jax.ShapeDtypeStruct((total, D), jnp.float32),
        mesh=mesh,
        scratch_types=[
            pltpu.VMEM((_NBUF, K, IDXW), jnp.int32),
            pltpu.VMEM((_NBUF, chunk, D), jnp.float32),
            pltpu.VMEM((chunk, D), jnp.float32),
        ]
        + [pltpu.SemaphoreType.DMA] * (3 * _NBUF),
        compiler_params=pltpu.CompilerParams(use_tc_tiling_on_sc=False),
    )
    def sc_kernel(x_hbm, table_hbm, pe_hbm, out_hbm, idx_v, bufs, pe_v, *sems):
        isem = sems[:_NBUF]
        gsem = sems[_NBUF : 2 * _NBUF]
        osem = sems[2 * _NBUF :]
        wid = lax.axis_index("s") * 2 + lax.axis_index("c")
        pltpu.sync_copy(pe_hbm, pe_v)

        def idx_copy(c, b):
            return pltpu.async_copy(x_hbm.at[wid, c], idx_v.at[b], isem[b])

        def gather_start(b):
            for j in range(K):
                pltpu.async_copy(
                    table_hbm.at[idx_v.at[b, j]],
                    bufs.at[b, pl.ds(j * IDXW, IDXW)],
                    gsem[b],
                )

        def gather_wait(b):
            for j in range(K):
                pltpu.make_async_copy(
                    table_hbm.at[idx_v.at[b, j]],
                    bufs.at[b, pl.ds(j * IDXW, IDXW)],
                    gsem[b],
                ).wait()

        def out_wait(b):
            pltpu.make_async_copy(
                bufs.at[b], out_hbm.at[pl.ds(0, chunk)], osem[b]
            ).wait()

        for b in range(_NBUF):
            idx_copy(b, b)
        for b in range(2):
            pltpu.make_async_copy(x_hbm.at[wid, b], idx_v.at[b], isem[b]).wait()
            gather_start(b)

        def group_body(g, carry):
            for b in range(_NBUF):
                s = _NBUF * g + b
                gather_wait(b)
                def row_body(r, rc):
                    for j in range(D // _NLANES):
                        sl = pl.ds(j * _NLANES, _NLANES)
                        bufs[b, r, sl] = bufs[b, r, sl] * _SCALE + pe_v[r, sl]
                    return rc

                lax.fori_loop(0, chunk, row_body, 0, unroll=4)
                r0 = wid * per_w + s * chunk
                pltpu.async_copy(bufs.at[b], out_hbm.at[pl.ds(r0, chunk)], osem[b])
                @pl.when(s + _NBUF < n_chunks)
                def _():
                    idx_copy(s + _NBUF, b)

                bp = (b + 2) % _NBUF

                @pl.when(s + 2 < n_chunks)
                def _():
                    if b < 2:

                        @pl.when(g > 0)
                        def _():
                            out_wait(bp)

                    else:
                        out_wait(bp)
                    pltpu.make_async_copy(
                        x_hbm.at[wid, s + 2], idx_v.at[bp], isem[bp]
                    ).wait()
                    gather_start(bp)

            return carry

        lax.fori_loop(0, n_groups, group_body, 0)
        for b in range(_NBUF):
            out_wait(b)

    return sc_kernel(x_r, table, pe)


def kernel(x, table):
    B, S = x.shape
    V, D = table.shape
    NW = 32
    chunk = S
    IDXW = 100
    K = chunk // IDXW
    per_w = (B * S) // NW
    n_chunks = per_w // chunk

    pe = np.zeros((S, D_MODEL), dtype=np.float32)
    pos = np.arange(S, dtype=np.float32)[:, None]
    div_term = np.exp(
        np.arange(0, D_MODEL, 2, dtype=np.float32) * (-math.log(10000.0) / D_MODEL)
    )
    pe[:, 0::2] = np.sin(pos * div_term)
    pe[:, 1::2] = np.cos(pos * div_term)

    x_r = x.reshape(NW, n_chunks, K, IDXW)
    out = _run(x_r, table, jnp.asarray(pe))
    return out.reshape(B, S, D)
